# Initial kernel scaffold; baseline (speedup 1.0000x reference)
import jax
import jax.numpy as jnp
from jax.experimental import pallas as pl


def kernel(x, emb0, emb1, emb2, emb3, emb4, emb5, emb6, emb7, emb8, W, b):
    raise NotImplementedError("write your pallas kernel here")


# speedup vs baseline: 5.7798x; 5.7798x over previous
import functools
import jax
import jax.numpy as jnp
from jax.experimental import pallas as pl

_CAT_DIMS = [119, 5, 12, 12, 10, 6, 6, 2, 2]
_OFFS = [0, 119, 124, 136, 148, 158, 164, 170, 172]
_TOT = 174
_WPAD = 256
_EMB = 64
_NCAT = 9
_ADD = 32
_B = 2000


def _fold_body(tt_ref, w1_ref, o_ref):
    o_ref[...] = jnp.dot(tt_ref[...], w1_ref[...],
                         preferred_element_type=jnp.float32)


def _main_body(x_ref, ttf_ref, w2_ref, b_ref, o_ref):
    x = x_ref[...]
    cols = jax.lax.broadcasted_iota(jnp.int32, (_B, _WPAD), 1)
    xi = x[:, :_NCAT].astype(jnp.int32)
    m = None
    for i in range(_NCAT):
        eq = (xi[:, i:i + 1] + _OFFS[i]) == cols
        m = eq if m is None else jnp.logical_or(m, eq)
    mhot = m.astype(jnp.float32)
    acc = jnp.dot(mhot, ttf_ref[...], preferred_element_type=jnp.float32)
    acc = acc + jnp.dot(x[:, _NCAT:], w2_ref[...],
                        preferred_element_type=jnp.float32)
    o_ref[...] = acc + b_ref[...]


def kernel(x, emb0, emb1, emb2, emb3, emb4, emb5, emb6, emb7, emb8, W, b):
    tables = [emb0, emb1, emb2, emb3, emb4, emb5, emb6, emb7, emb8]
    n = x.shape[0]
    tt = jnp.concatenate(
        tables + [jnp.zeros((_WPAD - _TOT, _EMB), jnp.float32)], axis=0)
    w1 = W[:_EMB]
    w2 = W[_EMB:]
    ttf = pl.pallas_call(
        _fold_body,
        out_shape=jax.ShapeDtypeStruct((_WPAD, _EMB), jnp.float32),
    )(tt, w1)
    out = pl.pallas_call(
        _main_body,
        grid=(n // _B,),
        in_specs=[
            pl.BlockSpec((_B, x.shape[1]), lambda i: (i, 0)),
            pl.BlockSpec((_WPAD, _EMB), lambda i: (0, 0)),
            pl.BlockSpec((_ADD, _EMB), lambda i: (0, 0)),
            pl.BlockSpec((1, _EMB), lambda i: (0, 0)),
        ],
        out_specs=pl.BlockSpec((_B, _EMB), lambda i: (i, 0)),
        out_shape=jax.ShapeDtypeStruct((n, _EMB), jnp.float32),
    )(x, ttf, w2, b.reshape(1, _EMB))
    return out
